# Initial kernel scaffold; baseline (speedup 1.0000x reference)
import jax
import jax.numpy as jnp
from jax.experimental import pallas as pl


def kernel(xyz, features, W1, gamma1, beta1, W2, gamma2, beta2):
    raise NotImplementedError("write your pallas kernel here")


# speedup vs baseline: 3.2737x; 3.2737x over previous
import functools

import jax
import jax.numpy as jnp
from jax import lax
from jax.experimental import pallas as pl
from jax.experimental.pallas import tpu as pltpu
from jax.experimental.pallas import tpu_sc as plsc

_B, _N, _K = 4, 4096, 16
_NK = _N * _K
_TQ = 256
_CH = 8192
_EPS = 1e-5

_NW = 32
_PW = (_B * _N) // _NW
_WPB = _N // _PW


def _knn_body(q_ref, c_ref, idx_ref):
    q = q_ref[0]
    c = c_ref[0]
    dot = lax.dot_general(q, c, (((1,), (0,)), ((), ())),
                          preferred_element_type=jnp.float32)
    sq_q = jnp.sum(q * q, axis=1, keepdims=True)
    sq_c = jnp.sum(c * c, axis=0, keepdims=True)
    dist = (sq_q - 2.0 * dot) + sq_c
    lane = lax.broadcasted_iota(jnp.int32, (_TQ, _N), 1)
    kl = lax.broadcasted_iota(jnp.int32, (_TQ, _K), 1)
    acc = jnp.zeros((_TQ, _K), jnp.int32)
    big = jnp.int32(2 ** 30)
    for k in range(_K):
        m = jnp.min(dist, axis=1, keepdims=True)
        hit = dist <= m
        ik = jnp.min(jnp.where(hit, lane, big), axis=1)
        acc = jnp.where(kl == k, ik[:, None], acc)
        dist = jnp.where(hit, jnp.inf, dist)
    idx_ref[0] = acc


def _knn(xyz, xyzt):
    return pl.pallas_call(
        _knn_body,
        grid=(_B, _N // _TQ),
        in_specs=[
            pl.BlockSpec((1, _TQ, 3), lambda b, t: (b, t, 0)),
            pl.BlockSpec((1, 3, _N), lambda b, t: (b, 0, 0)),
        ],
        out_specs=pl.BlockSpec((1, _TQ, _K), lambda b, t: (b, t, 0)),
        out_shape=jax.ShapeDtypeStruct((_B, _N, _K), jnp.int32),
    )(xyz, xyzt)


@functools.cache
def _make_sc_build():
    mesh = plsc.VectorSubcoreMesh(core_axis_name="c", subcore_axis_name="s")

    pwk = _PW * _K

    @functools.partial(
        pl.kernel,
        out_type=jax.ShapeDtypeStruct((_B * 10 * _NK,), jnp.float32),
        mesh=mesh,
        scratch_types=[
            pltpu.VMEM((_N,), jnp.float32),
            pltpu.VMEM((_N,), jnp.float32),
            pltpu.VMEM((_N,), jnp.float32),
            pltpu.VMEM((pwk,), jnp.int32),
            pltpu.VMEM((10 * pwk,), jnp.float32),
        ],
        compiler_params=pltpu.CompilerParams(needs_layout_passes=False),
    )
    def sc_build(xyzt_hbm, idx_hbm, s_hbm, xv, yv, zv, iv, sv):
        wid = lax.axis_index("s") * 2 + lax.axis_index("c")
        b = wid // _WPB
        n0 = (wid % _WPB) * _PW
        pltpu.sync_copy(xyzt_hbm.at[pl.ds((b * 3 + 0) * _N, _N)], xv)
        pltpu.sync_copy(xyzt_hbm.at[pl.ds((b * 3 + 1) * _N, _N)], yv)
        pltpu.sync_copy(xyzt_hbm.at[pl.ds((b * 3 + 2) * _N, _N)], zv)
        pltpu.sync_copy(idx_hbm.at[pl.ds(b * _NK + n0 * _K, pwk)], iv)

        def body(i, carry):
            qi = jnp.full((16,), n0 + i, jnp.int32)
            qx = plsc.load_gather(xv, [qi])
            qy = plsc.load_gather(yv, [qi])
            qz = plsc.load_gather(zv, [qi])
            ii = iv[pl.ds(i * _K, _K)]
            nx = plsc.load_gather(xv, [ii])
            ny = plsc.load_gather(yv, [ii])
            nz = plsc.load_gather(zv, [ii])
            rx = nx - qx
            ry = ny - qy
            rz = nz - qz
            d = rx * rx + ry * ry + rz * rz
            o = i * _K
            for c, vec in enumerate(
                    (qx, qy, qz, nx, ny, nz, rx, ry, rz, d)):
                sv[pl.ds(c * pwk + o, _K)] = vec
            return carry

        lax.fori_loop(0, _PW, body, 0)
        for c in range(10):
            pltpu.sync_copy(
                sv.at[pl.ds(c * pwk, pwk)],
                s_hbm.at[pl.ds((b * 10 + c) * _NK + n0 * _K, pwk)])

    return sc_build


def _matmul(w_ref, x):
    return lax.dot_general(
        w_ref[...], x, (((1,), (0,)), ((), ())),
        preferred_element_type=jnp.float32)


def _mom1_body(s_ref, w1_ref, out_ref):
    @pl.when((pl.program_id(0) == 0) & (pl.program_id(1) == 0))
    def _init():
        out_ref[...] = jnp.zeros_like(out_ref)

    x1 = _matmul(w1_ref, s_ref[0])
    ssum = jnp.sum(x1, axis=1, keepdims=True)
    ssq = jnp.sum(x1 * x1, axis=1, keepdims=True)
    out_ref[...] += jnp.concatenate([ssum, ssq], axis=1)


def _mom2_body(s_ref, w1_ref, w2_ref, a1_ref, b1_ref, out_ref):
    @pl.when((pl.program_id(0) == 0) & (pl.program_id(1) == 0))
    def _init():
        out_ref[...] = jnp.zeros_like(out_ref)

    x1 = _matmul(w1_ref, s_ref[0])
    h = jnp.maximum(x1 * a1_ref[...] + b1_ref[...], 0.0)
    x2 = _matmul(w2_ref, h)
    ssum = jnp.sum(x2, axis=1, keepdims=True)
    ssq = jnp.sum(x2 * x2, axis=1, keepdims=True)
    out_ref[...] += jnp.concatenate([ssum, ssq], axis=1)


def _final_body(s_ref, w1_ref, w2_ref, a1_ref, b1_ref, a2_ref, b2_ref,
                out_ref):
    x1 = _matmul(w1_ref, s_ref[0])
    h = jnp.maximum(x1 * a1_ref[...] + b1_ref[...], 0.0)
    x2 = _matmul(w2_ref, h)
    out_ref[0] = jnp.maximum(x2 * a2_ref[...] + b2_ref[...], 0.0)


_s_spec = pl.BlockSpec((1, 10, _CH), lambda b, j: (b, 0, j))
_w1_spec = pl.BlockSpec((32, 10), lambda b, j: (0, 0))
_w2_spec = pl.BlockSpec((32, 32), lambda b, j: (0, 0))
_v_spec = pl.BlockSpec((32, 1), lambda b, j: (0, 0))
_mom_spec = pl.BlockSpec((32, 2), lambda b, j: (0, 0))
_mom_shape = jax.ShapeDtypeStruct((32, 2), jnp.float32)
_grid = (_B, _NK // _CH)


def _mom1(s, w1):
    return pl.pallas_call(
        _mom1_body, grid=_grid,
        in_specs=[_s_spec, _w1_spec],
        out_specs=_mom_spec, out_shape=_mom_shape,
    )(s, w1)


def _mom2(s, w1, w2, a1, b1):
    return pl.pallas_call(
        _mom2_body, grid=_grid,
        in_specs=[_s_spec, _w1_spec, _w2_spec, _v_spec, _v_spec],
        out_specs=_mom_spec, out_shape=_mom_shape,
    )(s, w1, w2, a1, b1)


def _final(s, w1, w2, a1, b1, a2, b2):
    return pl.pallas_call(
        _final_body, grid=_grid,
        in_specs=[_s_spec, _w1_spec, _w2_spec, _v_spec, _v_spec, _v_spec,
                  _v_spec],
        out_specs=pl.BlockSpec((1, 32, _CH), lambda b, j: (b, 0, j)),
        out_shape=jax.ShapeDtypeStruct((_B, 32, _NK), jnp.float32),
    )(s, w1, w2, a1, b1, a2, b2)


def _bn_coeffs(m, gamma, beta):
    cnt = jnp.float32(_B * _NK)
    mean = m[:, 0] / cnt
    var = m[:, 1] / cnt - mean * mean
    r = gamma / jnp.sqrt(var + _EPS)
    return r[:, None], (beta - mean * r)[:, None]


def kernel(xyz, features, W1, gamma1, beta1, W2, gamma2, beta2):
    del features
    xyzt = jnp.transpose(xyz, (0, 2, 1))
    idx = _knn(xyz, xyzt)
    if True:
        bidx = jnp.arange(_B)[:, None, None]
        nb = xyz[bidx, idx]
        q = jnp.broadcast_to(xyz[:, :, None, :], nb.shape)
        rel = nb - q
        d = jnp.sum(rel ** 2, axis=-1, keepdims=True)
        s_bnkc = jnp.concatenate([q, nb, rel, d], axis=-1)
        s = jnp.transpose(s_bnkc.reshape(_B, _NK, 10), (0, 2, 1))
    else:
        s = _make_sc_build()(xyzt.reshape(-1), idx.reshape(-1))
        s = s.reshape(_B, 10, _NK)
    a1, b1 = _bn_coeffs(_mom1(s, W1), gamma1, beta1)
    a2, b2 = _bn_coeffs(_mom2(s, W1, W2, a1, b1), gamma2, beta2)
    out = _final(s, W1, W2, a1, b1, a2, b2)
    return out.reshape(_B, 32, _N, _K)


# speedup vs baseline: 14.9019x; 4.5520x over previous
import functools

import jax
import jax.numpy as jnp
from jax import lax
from jax.experimental import pallas as pl
from jax.experimental.pallas import tpu as pltpu
from jax.experimental.pallas import tpu_sc as plsc

_B, _N, _K = 4, 4096, 16
_NK = _N * _K
_TQ = 256
_CH = 8192
_EPS = 1e-5

_NW = 32
_PW = (_B * _N) // _NW
_WPB = _N // _PW


def _knn_body(q_ref, c_ref, idx_ref):
    q = q_ref[0]
    c = c_ref[0]
    dot = lax.dot_general(q, c, (((1,), (0,)), ((), ())),
                          preferred_element_type=jnp.float32)
    sq_q = jnp.sum(q * q, axis=1, keepdims=True)
    sq_c = jnp.sum(c * c, axis=0, keepdims=True)
    dist = (sq_q - 2.0 * dot) + sq_c
    lane = lax.broadcasted_iota(jnp.int32, (_TQ, _N), 1)
    kl = lax.broadcasted_iota(jnp.int32, (_TQ, _K), 1)
    acc = jnp.zeros((_TQ, _K), jnp.int32)
    big = jnp.int32(2 ** 30)
    for k in range(_K):
        m = jnp.min(dist, axis=1, keepdims=True)
        hit = dist <= m
        ik = jnp.min(jnp.where(hit, lane, big), axis=1)
        acc = jnp.where(kl == k, ik[:, None], acc)
        dist = jnp.where(hit, jnp.inf, dist)
    idx_ref[0] = acc


def _knn(xyz, xyzt):
    return pl.pallas_call(
        _knn_body,
        grid=(_B, _N // _TQ),
        in_specs=[
            pl.BlockSpec((1, _TQ, 3), lambda b, t: (b, t, 0)),
            pl.BlockSpec((1, 3, _N), lambda b, t: (b, 0, 0)),
        ],
        out_specs=pl.BlockSpec((1, _TQ, _K), lambda b, t: (b, t, 0)),
        out_shape=jax.ShapeDtypeStruct((_B, _N, _K), jnp.int32),
    )(xyz, xyzt)


@functools.cache
def _make_sc_build():
    mesh = plsc.VectorSubcoreMesh(core_axis_name="c", subcore_axis_name="s")

    pwk = _PW * _K

    @functools.partial(
        pl.kernel,
        out_type=jax.ShapeDtypeStruct((_B * 10 * _NK,), jnp.float32),
        mesh=mesh,
        scratch_types=[
            pltpu.VMEM((_N,), jnp.float32),
            pltpu.VMEM((_N,), jnp.float32),
            pltpu.VMEM((_N,), jnp.float32),
            pltpu.VMEM((pwk,), jnp.int32),
            pltpu.VMEM((10 * pwk,), jnp.float32),
        ],
        compiler_params=pltpu.CompilerParams(needs_layout_passes=False),
    )
    def sc_build(xyzt_hbm, idx_hbm, s_hbm, xv, yv, zv, iv, sv):
        wid = lax.axis_index("s") * 2 + lax.axis_index("c")
        b = wid // _WPB
        n0 = (wid % _WPB) * _PW
        pltpu.sync_copy(xyzt_hbm.at[pl.ds((b * 3 + 0) * _N, _N)], xv)
        pltpu.sync_copy(xyzt_hbm.at[pl.ds((b * 3 + 1) * _N, _N)], yv)
        pltpu.sync_copy(xyzt_hbm.at[pl.ds((b * 3 + 2) * _N, _N)], zv)
        pltpu.sync_copy(idx_hbm.at[pl.ds(b * _NK + n0 * _K, pwk)], iv)

        def body(i, carry):
            qi = jnp.full((16,), n0 + i, jnp.int32)
            qx = plsc.load_gather(xv, [qi])
            qy = plsc.load_gather(yv, [qi])
            qz = plsc.load_gather(zv, [qi])
            ii = iv[pl.ds(i * _K, _K)]
            nx = plsc.load_gather(xv, [ii])
            ny = plsc.load_gather(yv, [ii])
            nz = plsc.load_gather(zv, [ii])
            rx = nx - qx
            ry = ny - qy
            rz = nz - qz
            d = rx * rx + ry * ry + rz * rz
            o = i * _K
            for c, vec in enumerate(
                    (qx, qy, qz, nx, ny, nz, rx, ry, rz, d)):
                sv[pl.ds(c * pwk + o, _K)] = vec
            return carry

        lax.fori_loop(0, _PW, body, 0)
        for c in range(10):
            pltpu.sync_copy(
                sv.at[pl.ds(c * pwk, pwk)],
                s_hbm.at[pl.ds((b * 10 + c) * _NK + n0 * _K, pwk)])

    return sc_build


def _matmul(w_ref, x):
    return lax.dot_general(
        w_ref[...], x, (((1,), (0,)), ((), ())),
        preferred_element_type=jnp.float32)


def _mom1_body(s_ref, w1_ref, out_ref):
    @pl.when((pl.program_id(0) == 0) & (pl.program_id(1) == 0))
    def _init():
        out_ref[...] = jnp.zeros_like(out_ref)

    x1 = _matmul(w1_ref, s_ref[0])
    ssum = jnp.sum(x1, axis=1, keepdims=True)
    ssq = jnp.sum(x1 * x1, axis=1, keepdims=True)
    out_ref[...] += jnp.concatenate([ssum, ssq], axis=1)


def _mom2_body(s_ref, w1_ref, w2_ref, a1_ref, b1_ref, out_ref):
    @pl.when((pl.program_id(0) == 0) & (pl.program_id(1) == 0))
    def _init():
        out_ref[...] = jnp.zeros_like(out_ref)

    x1 = _matmul(w1_ref, s_ref[0])
    h = jnp.maximum(x1 * a1_ref[...] + b1_ref[...], 0.0)
    x2 = _matmul(w2_ref, h)
    ssum = jnp.sum(x2, axis=1, keepdims=True)
    ssq = jnp.sum(x2 * x2, axis=1, keepdims=True)
    out_ref[...] += jnp.concatenate([ssum, ssq], axis=1)


def _final_body(s_ref, w1_ref, w2_ref, a1_ref, b1_ref, a2_ref, b2_ref,
                out_ref):
    x1 = _matmul(w1_ref, s_ref[0])
    h = jnp.maximum(x1 * a1_ref[...] + b1_ref[...], 0.0)
    x2 = _matmul(w2_ref, h)
    out_ref[0] = jnp.maximum(x2 * a2_ref[...] + b2_ref[...], 0.0)


_s_spec = pl.BlockSpec((1, 10, _CH), lambda b, j: (b, 0, j))
_w1_spec = pl.BlockSpec((32, 10), lambda b, j: (0, 0))
_w2_spec = pl.BlockSpec((32, 32), lambda b, j: (0, 0))
_v_spec = pl.BlockSpec((32, 1), lambda b, j: (0, 0))
_mom_spec = pl.BlockSpec((32, 2), lambda b, j: (0, 0))
_mom_shape = jax.ShapeDtypeStruct((32, 2), jnp.float32)
_grid = (_B, _NK // _CH)


def _mom1(s, w1):
    return pl.pallas_call(
        _mom1_body, grid=_grid,
        in_specs=[_s_spec, _w1_spec],
        out_specs=_mom_spec, out_shape=_mom_shape,
    )(s, w1)


def _mom2(s, w1, w2, a1, b1):
    return pl.pallas_call(
        _mom2_body, grid=_grid,
        in_specs=[_s_spec, _w1_spec, _w2_spec, _v_spec, _v_spec],
        out_specs=_mom_spec, out_shape=_mom_shape,
    )(s, w1, w2, a1, b1)


def _final(s, w1, w2, a1, b1, a2, b2):
    return pl.pallas_call(
        _final_body, grid=_grid,
        in_specs=[_s_spec, _w1_spec, _w2_spec, _v_spec, _v_spec, _v_spec,
                  _v_spec],
        out_specs=pl.BlockSpec((1, 32, _CH), lambda b, j: (b, 0, j)),
        out_shape=jax.ShapeDtypeStruct((_B, 32, _NK), jnp.float32),
    )(s, w1, w2, a1, b1, a2, b2)


def _bn_coeffs(m, gamma, beta):
    cnt = jnp.float32(_B * _NK)
    mean = m[:, 0] / cnt
    var = m[:, 1] / cnt - mean * mean
    r = gamma / jnp.sqrt(var + _EPS)
    return r[:, None], (beta - mean * r)[:, None]


def kernel(xyz, features, W1, gamma1, beta1, W2, gamma2, beta2):
    del features
    xyzt = jnp.transpose(xyz, (0, 2, 1))
    idx = _knn(xyz, xyzt)
    s = _make_sc_build()(xyzt.reshape(-1), idx.reshape(-1))
    s = s.reshape(_B, 10, _NK)
    a1, b1 = _bn_coeffs(_mom1(s, W1), gamma1, beta1)
    a2, b2 = _bn_coeffs(_mom2(s, W1, W2, a1, b1), gamma2, beta2)
    out = _final(s, W1, W2, a1, b1, a2, b2)
    return out.reshape(_B, 32, _N, _K)


# speedup vs baseline: 15.0438x; 1.0095x over previous
import functools

import jax
import jax.numpy as jnp
from jax import lax
from jax.experimental import pallas as pl
from jax.experimental.pallas import tpu as pltpu
from jax.experimental.pallas import tpu_sc as plsc

_B, _N, _K = 4, 4096, 16
_NK = _N * _K
_TQ = 256
_CH = 8192
_EPS = 1e-5

_NW = 32
_PW = (_B * _N) // _NW
_WPB = _N // _PW


def _knn_body(q_ref, c_ref, idx_ref):
    q = q_ref[0]
    c = c_ref[0]
    dot = lax.dot_general(q, c, (((1,), (0,)), ((), ())),
                          preferred_element_type=jnp.float32)
    sq_q = jnp.sum(q * q, axis=1, keepdims=True)
    sq_c = jnp.sum(c * c, axis=0, keepdims=True)
    dist = (sq_q - 2.0 * dot) + sq_c
    lane = lax.broadcasted_iota(jnp.int32, (_TQ, _N), 1)
    kl = lax.broadcasted_iota(jnp.int32, (_TQ, _K), 1)
    acc = jnp.zeros((_TQ, _K), jnp.int32)
    for k in range(_K):
        ik = jnp.argmin(dist, axis=1)
        acc = jnp.where(kl == k, ik[:, None], acc)
        dist = jnp.where(lane == ik[:, None], jnp.inf, dist)
    idx_ref[0] = acc


def _knn(xyz, xyzt):
    return pl.pallas_call(
        _knn_body,
        grid=(_B, _N // _TQ),
        in_specs=[
            pl.BlockSpec((1, _TQ, 3), lambda b, t: (b, t, 0)),
            pl.BlockSpec((1, 3, _N), lambda b, t: (b, 0, 0)),
        ],
        out_specs=pl.BlockSpec((1, _TQ, _K), lambda b, t: (b, t, 0)),
        out_shape=jax.ShapeDtypeStruct((_B, _N, _K), jnp.int32),
    )(xyz, xyzt)


@functools.cache
def _make_sc_build():
    mesh = plsc.VectorSubcoreMesh(core_axis_name="c", subcore_axis_name="s")

    pwk = _PW * _K

    @functools.partial(
        pl.kernel,
        out_type=jax.ShapeDtypeStruct((_B * 10 * _NK,), jnp.float32),
        mesh=mesh,
        scratch_types=[
            pltpu.VMEM((_N,), jnp.float32),
            pltpu.VMEM((_N,), jnp.float32),
            pltpu.VMEM((_N,), jnp.float32),
            pltpu.VMEM((pwk,), jnp.int32),
            pltpu.VMEM((10 * pwk,), jnp.float32),
        ],
        compiler_params=pltpu.CompilerParams(needs_layout_passes=False),
    )
    def sc_build(xyzt_hbm, idx_hbm, s_hbm, xv, yv, zv, iv, sv):
        wid = lax.axis_index("s") * 2 + lax.axis_index("c")
        b = wid // _WPB
        n0 = (wid % _WPB) * _PW
        pltpu.sync_copy(xyzt_hbm.at[pl.ds((b * 3 + 0) * _N, _N)], xv)
        pltpu.sync_copy(xyzt_hbm.at[pl.ds((b * 3 + 1) * _N, _N)], yv)
        pltpu.sync_copy(xyzt_hbm.at[pl.ds((b * 3 + 2) * _N, _N)], zv)
        pltpu.sync_copy(idx_hbm.at[pl.ds(b * _NK + n0 * _K, pwk)], iv)

        def body(i, carry):
            qi = jnp.full((16,), n0 + i, jnp.int32)
            qx = plsc.load_gather(xv, [qi])
            qy = plsc.load_gather(yv, [qi])
            qz = plsc.load_gather(zv, [qi])
            ii = iv[pl.ds(i * _K, _K)]
            nx = plsc.load_gather(xv, [ii])
            ny = plsc.load_gather(yv, [ii])
            nz = plsc.load_gather(zv, [ii])
            rx = nx - qx
            ry = ny - qy
            rz = nz - qz
            d = rx * rx + ry * ry + rz * rz
            o = i * _K
            for c, vec in enumerate(
                    (qx, qy, qz, nx, ny, nz, rx, ry, rz, d)):
                sv[pl.ds(c * pwk + o, _K)] = vec
            return carry

        lax.fori_loop(0, _PW, body, 0)
        for c in range(10):
            pltpu.sync_copy(
                sv.at[pl.ds(c * pwk, pwk)],
                s_hbm.at[pl.ds((b * 10 + c) * _NK + n0 * _K, pwk)])

    return sc_build


def _matmul(w_ref, x):
    return lax.dot_general(
        w_ref[...], x, (((1,), (0,)), ((), ())),
        preferred_element_type=jnp.float32)


def _mom1_body(s_ref, w1_ref, out_ref):
    @pl.when((pl.program_id(0) == 0) & (pl.program_id(1) == 0))
    def _init():
        out_ref[...] = jnp.zeros_like(out_ref)

    x1 = _matmul(w1_ref, s_ref[0])
    ssum = jnp.sum(x1, axis=1, keepdims=True)
    ssq = jnp.sum(x1 * x1, axis=1, keepdims=True)
    out_ref[...] += jnp.concatenate([ssum, ssq], axis=1)


def _mom2_body(s_ref, w1_ref, w2_ref, a1_ref, b1_ref, out_ref):
    @pl.when((pl.program_id(0) == 0) & (pl.program_id(1) == 0))
    def _init():
        out_ref[...] = jnp.zeros_like(out_ref)

    x1 = _matmul(w1_ref, s_ref[0])
    h = jnp.maximum(x1 * a1_ref[...] + b1_ref[...], 0.0)
    x2 = _matmul(w2_ref, h)
    ssum = jnp.sum(x2, axis=1, keepdims=True)
    ssq = jnp.sum(x2 * x2, axis=1, keepdims=True)
    out_ref[...] += jnp.concatenate([ssum, ssq], axis=1)


def _final_body(s_ref, w1_ref, w2_ref, a1_ref, b1_ref, a2_ref, b2_ref,
                out_ref):
    x1 = _matmul(w1_ref, s_ref[0])
    h = jnp.maximum(x1 * a1_ref[...] + b1_ref[...], 0.0)
    x2 = _matmul(w2_ref, h)
    out_ref[0] = jnp.maximum(x2 * a2_ref[...] + b2_ref[...], 0.0)


_s_spec = pl.BlockSpec((1, 10, _CH), lambda b, j: (b, 0, j))
_w1_spec = pl.BlockSpec((32, 10), lambda b, j: (0, 0))
_w2_spec = pl.BlockSpec((32, 32), lambda b, j: (0, 0))
_v_spec = pl.BlockSpec((32, 1), lambda b, j: (0, 0))
_mom_spec = pl.BlockSpec((32, 2), lambda b, j: (0, 0))
_mom_shape = jax.ShapeDtypeStruct((32, 2), jnp.float32)
_grid = (_B, _NK // _CH)


def _mom1(s, w1):
    return pl.pallas_call(
        _mom1_body, grid=_grid,
        in_specs=[_s_spec, _w1_spec],
        out_specs=_mom_spec, out_shape=_mom_shape,
    )(s, w1)


def _mom2(s, w1, w2, a1, b1):
    return pl.pallas_call(
        _mom2_body, grid=_grid,
        in_specs=[_s_spec, _w1_spec, _w2_spec, _v_spec, _v_spec],
        out_specs=_mom_spec, out_shape=_mom_shape,
    )(s, w1, w2, a1, b1)


def _final(s, w1, w2, a1, b1, a2, b2):
    return pl.pallas_call(
        _final_body, grid=_grid,
        in_specs=[_s_spec, _w1_spec, _w2_spec, _v_spec, _v_spec, _v_spec,
                  _v_spec],
        out_specs=pl.BlockSpec((1, 32, _CH), lambda b, j: (b, 0, j)),
        out_shape=jax.ShapeDtypeStruct((_B, 32, _NK), jnp.float32),
    )(s, w1, w2, a1, b1, a2, b2)


def _bn_coeffs(m, gamma, beta):
    cnt = jnp.float32(_B * _NK)
    mean = m[:, 0] / cnt
    var = m[:, 1] / cnt - mean * mean
    r = gamma / jnp.sqrt(var + _EPS)
    return r[:, None], (beta - mean * r)[:, None]


def kernel(xyz, features, W1, gamma1, beta1, W2, gamma2, beta2):
    del features
    xyzt = jnp.transpose(xyz, (0, 2, 1))
    idx = _knn(xyz, xyzt)
    s = _make_sc_build()(xyzt.reshape(-1), idx.reshape(-1))
    s = s.reshape(_B, 10, _NK)
    a1, b1 = _bn_coeffs(_mom1(s, W1), gamma1, beta1)
    a2, b2 = _bn_coeffs(_mom2(s, W1, W2, a1, b1), gamma2, beta2)
    out = _final(s, W1, W2, a1, b1, a2, b2)
    return out.reshape(_B, 32, _N, _K)


# speedup vs baseline: 15.3851x; 1.0227x over previous
import functools

import jax
import jax.numpy as jnp
from jax import lax
from jax.experimental import pallas as pl
from jax.experimental.pallas import tpu as pltpu
from jax.experimental.pallas import tpu_sc as plsc

_B, _N, _K = 4, 4096, 16
_NK = _N * _K
_TQ = 512
_CH = 8192
_EPS = 1e-5

_NW = 32
_PW = (_B * _N) // _NW
_WPB = _N // _PW


def _knn_body(q_ref, c_ref, idx_ref):
    q = q_ref[0]
    c = c_ref[0]
    dot = lax.dot_general(q, c, (((1,), (0,)), ((), ())),
                          preferred_element_type=jnp.float32)
    sq_q = jnp.sum(q * q, axis=1, keepdims=True)
    sq_c = jnp.sum(c * c, axis=0, keepdims=True)
    dist = (sq_q - 2.0 * dot) + sq_c
    lane = lax.broadcasted_iota(jnp.int32, (_TQ, _N), 1)
    kl = lax.broadcasted_iota(jnp.int32, (_TQ, _K), 1)
    acc = jnp.zeros((_TQ, _K), jnp.int32)
    for k in range(_K):
        ik = jnp.argmin(dist, axis=1)
        acc = jnp.where(kl == k, ik[:, None], acc)
        dist = jnp.where(lane == ik[:, None], jnp.inf, dist)
    idx_ref[0] = acc


def _knn(xyz, xyzt):
    return pl.pallas_call(
        _knn_body,
        grid=(_B, _N // _TQ),
        in_specs=[
            pl.BlockSpec((1, _TQ, 3), lambda b, t: (b, t, 0)),
            pl.BlockSpec((1, 3, _N), lambda b, t: (b, 0, 0)),
        ],
        out_specs=pl.BlockSpec((1, _TQ, _K), lambda b, t: (b, t, 0)),
        out_shape=jax.ShapeDtypeStruct((_B, _N, _K), jnp.int32),
    )(xyz, xyzt)


@functools.cache
def _make_sc_build():
    mesh = plsc.VectorSubcoreMesh(core_axis_name="c", subcore_axis_name="s")

    pwk = _PW * _K

    @functools.partial(
        pl.kernel,
        out_type=jax.ShapeDtypeStruct((_B * 10 * _NK,), jnp.float32),
        mesh=mesh,
        scratch_types=[
            pltpu.VMEM((_N,), jnp.float32),
            pltpu.VMEM((_N,), jnp.float32),
            pltpu.VMEM((_N,), jnp.float32),
            pltpu.VMEM((pwk,), jnp.int32),
            pltpu.VMEM((10 * pwk,), jnp.float32),
        ],
        compiler_params=pltpu.CompilerParams(needs_layout_passes=False),
    )
    def sc_build(xyzt_hbm, idx_hbm, s_hbm, xv, yv, zv, iv, sv):
        wid = lax.axis_index("s") * 2 + lax.axis_index("c")
        b = wid // _WPB
        n0 = (wid % _WPB) * _PW
        pltpu.sync_copy(xyzt_hbm.at[pl.ds((b * 3 + 0) * _N, _N)], xv)
        pltpu.sync_copy(xyzt_hbm.at[pl.ds((b * 3 + 1) * _N, _N)], yv)
        pltpu.sync_copy(xyzt_hbm.at[pl.ds((b * 3 + 2) * _N, _N)], zv)
        pltpu.sync_copy(idx_hbm.at[pl.ds(b * _NK + n0 * _K, pwk)], iv)

        def body(i, carry):
            qi = jnp.full((16,), n0 + i, jnp.int32)
            qx = plsc.load_gather(xv, [qi])
            qy = plsc.load_gather(yv, [qi])
            qz = plsc.load_gather(zv, [qi])
            ii = iv[pl.ds(i * _K, _K)]
            nx = plsc.load_gather(xv, [ii])
            ny = plsc.load_gather(yv, [ii])
            nz = plsc.load_gather(zv, [ii])
            rx = nx - qx
            ry = ny - qy
            rz = nz - qz
            d = rx * rx + ry * ry + rz * rz
            o = i * _K
            for c, vec in enumerate(
                    (qx, qy, qz, nx, ny, nz, rx, ry, rz, d)):
                sv[pl.ds(c * pwk + o, _K)] = vec
            return carry

        lax.fori_loop(0, _PW, body, 0)
        for c in range(10):
            pltpu.sync_copy(
                sv.at[pl.ds(c * pwk, pwk)],
                s_hbm.at[pl.ds((b * 10 + c) * _NK + n0 * _K, pwk)])

    return sc_build


def _matmul(w_ref, x):
    return lax.dot_general(
        w_ref[...], x, (((1,), (0,)), ((), ())),
        preferred_element_type=jnp.float32)


def _mom1_body(s_ref, w1_ref, out_ref):
    @pl.when((pl.program_id(0) == 0) & (pl.program_id(1) == 0))
    def _init():
        out_ref[...] = jnp.zeros_like(out_ref)

    x1 = _matmul(w1_ref, s_ref[0])
    ssum = jnp.sum(x1, axis=1, keepdims=True)
    ssq = jnp.sum(x1 * x1, axis=1, keepdims=True)
    out_ref[...] += jnp.concatenate([ssum, ssq], axis=1)


def _mom2_body(s_ref, w1_ref, w2_ref, a1_ref, b1_ref, out_ref):
    @pl.when((pl.program_id(0) == 0) & (pl.program_id(1) == 0))
    def _init():
        out_ref[...] = jnp.zeros_like(out_ref)

    x1 = _matmul(w1_ref, s_ref[0])
    h = jnp.maximum(x1 * a1_ref[...] + b1_ref[...], 0.0)
    x2 = _matmul(w2_ref, h)
    ssum = jnp.sum(x2, axis=1, keepdims=True)
    ssq = jnp.sum(x2 * x2, axis=1, keepdims=True)
    out_ref[...] += jnp.concatenate([ssum, ssq], axis=1)


def _final_body(s_ref, w1_ref, w2_ref, a1_ref, b1_ref, a2_ref, b2_ref,
                out_ref):
    x1 = _matmul(w1_ref, s_ref[0])
    h = jnp.maximum(x1 * a1_ref[...] + b1_ref[...], 0.0)
    x2 = _matmul(w2_ref, h)
    out_ref[0] = jnp.maximum(x2 * a2_ref[...] + b2_ref[...], 0.0)


_s_spec = pl.BlockSpec((1, 10, _CH), lambda b, j: (b, 0, j))
_w1_spec = pl.BlockSpec((32, 10), lambda b, j: (0, 0))
_w2_spec = pl.BlockSpec((32, 32), lambda b, j: (0, 0))
_v_spec = pl.BlockSpec((32, 1), lambda b, j: (0, 0))
_mom_spec = pl.BlockSpec((32, 2), lambda b, j: (0, 0))
_mom_shape = jax.ShapeDtypeStruct((32, 2), jnp.float32)
_grid = (_B, _NK // _CH)


def _mom1(s, w1):
    return pl.pallas_call(
        _mom1_body, grid=_grid,
        in_specs=[_s_spec, _w1_spec],
        out_specs=_mom_spec, out_shape=_mom_shape,
    )(s, w1)


def _mom2(s, w1, w2, a1, b1):
    return pl.pallas_call(
        _mom2_body, grid=_grid,
        in_specs=[_s_spec, _w1_spec, _w2_spec, _v_spec, _v_spec],
        out_specs=_mom_spec, out_shape=_mom_shape,
    )(s, w1, w2, a1, b1)


def _final(s, w1, w2, a1, b1, a2, b2):
    return pl.pallas_call(
        _final_body, grid=_grid,
        in_specs=[_s_spec, _w1_spec, _w2_spec, _v_spec, _v_spec, _v_spec,
                  _v_spec],
        out_specs=pl.BlockSpec((1, 32, _CH), lambda b, j: (b, 0, j)),
        out_shape=jax.ShapeDtypeStruct((_B, 32, _NK), jnp.float32),
    )(s, w1, w2, a1, b1, a2, b2)


def _bn_coeffs(m, gamma, beta):
    cnt = jnp.float32(_B * _NK)
    mean = m[:, 0] / cnt
    var = m[:, 1] / cnt - mean * mean
    r = gamma / jnp.sqrt(var + _EPS)
    return r[:, None], (beta - mean * r)[:, None]


def kernel(xyz, features, W1, gamma1, beta1, W2, gamma2, beta2):
    del features
    xyzt = jnp.transpose(xyz, (0, 2, 1))
    idx = _knn(xyz, xyzt)
    s = _make_sc_build()(xyzt.reshape(-1), idx.reshape(-1))
    s = s.reshape(_B, 10, _NK)
    a1, b1 = _bn_coeffs(_mom1(s, W1), gamma1, beta1)
    a2, b2 = _bn_coeffs(_mom2(s, W1, W2, a1, b1), gamma2, beta2)
    out = _final(s, W1, W2, a1, b1, a2, b2)
    return out.reshape(_B, 32, _N, _K)


# speedup vs baseline: 15.4197x; 1.0022x over previous
import functools

import jax
import jax.numpy as jnp
from jax import lax
from jax.experimental import pallas as pl
from jax.experimental.pallas import tpu as pltpu
from jax.experimental.pallas import tpu_sc as plsc

_B, _N, _K = 4, 4096, 16
_NK = _N * _K
_TQ = 512
_CH = 8192
_EPS = 1e-5

_NW = 32
_PW = (_B * _N) // _NW
_WPB = _N // _PW


def _knn_body(q_ref, c_ref, idx_ref):
    q = q_ref[0]
    c = c_ref[0]
    dot = lax.dot_general(q, c, (((1,), (0,)), ((), ())),
                          preferred_element_type=jnp.float32)
    sq_q = jnp.sum(q * q, axis=1, keepdims=True)
    sq_c = jnp.sum(c * c, axis=0, keepdims=True)
    dist = (sq_q - 2.0 * dot) + sq_c
    lane = lax.broadcasted_iota(jnp.int32, (_TQ, _N), 1)
    kl = lax.broadcasted_iota(jnp.int32, (_TQ, _K), 1)
    acc = jnp.zeros((_TQ, _K), jnp.int32)
    for k in range(_K):
        ik = jnp.argmin(dist, axis=1)
        acc = jnp.where(kl == k, ik[:, None], acc)
        dist = jnp.where(lane == ik[:, None], jnp.inf, dist)
    idx_ref[0] = acc


def _knn(xyz, xyzt):
    return pl.pallas_call(
        _knn_body,
        grid=(_B, _N // _TQ),
        in_specs=[
            pl.BlockSpec((1, _TQ, 3), lambda b, t: (b, t, 0)),
            pl.BlockSpec((1, 3, _N), lambda b, t: (b, 0, 0)),
        ],
        out_specs=pl.BlockSpec((1, _TQ, _K), lambda b, t: (b, t, 0)),
        out_shape=jax.ShapeDtypeStruct((_B, _N, _K), jnp.int32),
    )(xyz, xyzt)


@functools.cache
def _make_sc_build():
    mesh = plsc.VectorSubcoreMesh(core_axis_name="c", subcore_axis_name="s")

    pwk = _PW * _K

    @functools.partial(
        pl.kernel,
        out_type=jax.ShapeDtypeStruct((_B * 10 * _NK,), jnp.float32),
        mesh=mesh,
        scratch_types=[
            pltpu.VMEM((_N,), jnp.float32),
            pltpu.VMEM((_N,), jnp.float32),
            pltpu.VMEM((_N,), jnp.float32),
            pltpu.VMEM((pwk,), jnp.int32),
            pltpu.VMEM((10 * pwk,), jnp.float32),
        ],
        compiler_params=pltpu.CompilerParams(needs_layout_passes=False),
    )
    def sc_build(xyzt_hbm, idx_hbm, s_hbm, xv, yv, zv, iv, sv):
        wid = lax.axis_index("s") * 2 + lax.axis_index("c")
        b = wid // _WPB
        n0 = (wid % _WPB) * _PW
        pltpu.sync_copy(xyzt_hbm.at[pl.ds((b * 3 + 0) * _N, _N)], xv)
        pltpu.sync_copy(xyzt_hbm.at[pl.ds((b * 3 + 1) * _N, _N)], yv)
        pltpu.sync_copy(xyzt_hbm.at[pl.ds((b * 3 + 2) * _N, _N)], zv)
        pltpu.sync_copy(idx_hbm.at[pl.ds(b * _NK + n0 * _K, pwk)], iv)

        def body(i, carry):
            qi = jnp.full((16,), n0 + i, jnp.int32)
            qx = plsc.load_gather(xv, [qi])
            qy = plsc.load_gather(yv, [qi])
            qz = plsc.load_gather(zv, [qi])
            ii = iv[pl.ds(i * _K, _K)]
            nx = plsc.load_gather(xv, [ii])
            ny = plsc.load_gather(yv, [ii])
            nz = plsc.load_gather(zv, [ii])
            rx = nx - qx
            ry = ny - qy
            rz = nz - qz
            d = rx * rx + ry * ry + rz * rz
            o = i * _K
            for c, vec in enumerate(
                    (qx, qy, qz, nx, ny, nz, rx, ry, rz, d)):
                sv[pl.ds(c * pwk + o, _K)] = vec
            return carry

        lax.fori_loop(0, _PW, body, 0)
        for c in range(10):
            pltpu.sync_copy(
                sv.at[pl.ds(c * pwk, pwk)],
                s_hbm.at[pl.ds((b * 10 + c) * _NK + n0 * _K, pwk)])

    return sc_build


def _matmul(w_ref, x):
    return lax.dot_general(
        w_ref[...], x, (((1,), (0,)), ((), ())),
        preferred_element_type=jnp.float32)


def _mom1_body(s_ref, w1_ref, out_ref):
    @pl.when((pl.program_id(0) == 0) & (pl.program_id(1) == 0))
    def _init():
        out_ref[...] = jnp.zeros_like(out_ref)

    x1 = _matmul(w1_ref, s_ref[0])
    ssum = jnp.sum(x1, axis=1, keepdims=True)
    ssq = jnp.sum(x1 * x1, axis=1, keepdims=True)
    out_ref[...] += jnp.concatenate([ssum, ssq], axis=1)


def _mom2_body(s_ref, w1_ref, w2_ref, a1_ref, b1_ref, out_ref):
    @pl.when((pl.program_id(0) == 0) & (pl.program_id(1) == 0))
    def _init():
        out_ref[...] = jnp.zeros_like(out_ref)

    x1 = _matmul(w1_ref, s_ref[0])
    h = jnp.maximum(x1 * a1_ref[...] + b1_ref[...], 0.0)
    x2 = _matmul(w2_ref, h)
    ssum = jnp.sum(x2, axis=1, keepdims=True)
    ssq = jnp.sum(x2 * x2, axis=1, keepdims=True)
    out_ref[...] += jnp.concatenate([ssum, ssq], axis=1)


def _final_body(s_ref, w1_ref, w2_ref, a1_ref, b1_ref, a2_ref, b2_ref,
                out_ref):
    x1 = _matmul(w1_ref, s_ref[0])
    h = jnp.maximum(x1 * a1_ref[...] + b1_ref[...], 0.0)
    x2 = _matmul(w2_ref, h)
    y = jnp.maximum(x2 * a2_ref[...] + b2_ref[...], 0.0)
    out_ref[0] = y.reshape(32, _CH // _K, _K)


_s_spec = pl.BlockSpec((1, 10, _CH), lambda b, j: (b, 0, j))
_w1_spec = pl.BlockSpec((32, 10), lambda b, j: (0, 0))
_w2_spec = pl.BlockSpec((32, 32), lambda b, j: (0, 0))
_v_spec = pl.BlockSpec((32, 1), lambda b, j: (0, 0))
_mom_spec = pl.BlockSpec((32, 2), lambda b, j: (0, 0))
_mom_shape = jax.ShapeDtypeStruct((32, 2), jnp.float32)
_grid = (_B, _NK // _CH)


def _mom1(s, w1):
    return pl.pallas_call(
        _mom1_body, grid=_grid,
        in_specs=[_s_spec, _w1_spec],
        out_specs=_mom_spec, out_shape=_mom_shape,
    )(s, w1)


def _mom2(s, w1, w2, a1, b1):
    return pl.pallas_call(
        _mom2_body, grid=_grid,
        in_specs=[_s_spec, _w1_spec, _w2_spec, _v_spec, _v_spec],
        out_specs=_mom_spec, out_shape=_mom_shape,
    )(s, w1, w2, a1, b1)


def _final(s, w1, w2, a1, b1, a2, b2):
    return pl.pallas_call(
        _final_body, grid=_grid,
        in_specs=[_s_spec, _w1_spec, _w2_spec, _v_spec, _v_spec, _v_spec,
                  _v_spec],
        out_specs=pl.BlockSpec((1, 32, _CH // _K, _K), lambda b, j: (b, 0, j, 0)),
        out_shape=jax.ShapeDtypeStruct((_B, 32, _N, _K), jnp.float32),
    )(s, w1, w2, a1, b1, a2, b2)


def _bn_coeffs(m, gamma, beta):
    cnt = jnp.float32(_B * _NK)
    mean = m[:, 0] / cnt
    var = m[:, 1] / cnt - mean * mean
    r = gamma / jnp.sqrt(var + _EPS)
    return r[:, None], (beta - mean * r)[:, None]


def kernel(xyz, features, W1, gamma1, beta1, W2, gamma2, beta2):
    del features
    xyzt = jnp.transpose(xyz, (0, 2, 1))
    idx = _knn(xyz, xyzt)
    s = _make_sc_build()(xyzt.reshape(-1), idx.reshape(-1))
    s = s.reshape(_B, 10, _NK)
    a1, b1 = _bn_coeffs(_mom1(s, W1), gamma1, beta1)
    a2, b2 = _bn_coeffs(_mom2(s, W1, W2, a1, b1), gamma2, beta2)
    return _final(s, W1, W2, a1, b1, a2, b2)
